# Initial kernel scaffold; baseline (speedup 1.0000x reference)
import jax
import jax.numpy as jnp
from jax.experimental import pallas as pl


def kernel(x, edge_index, batch, W1, b1, W2, b2, Wg, bg, Wf, bf):
    raise NotImplementedError("write your pallas kernel here")


# speedup vs baseline: 10.6236x; 10.6236x over previous
import functools

import jax
import jax.numpy as jnp
from jax import lax
from jax.experimental import pallas as pl
from jax.experimental.pallas import tpu as pltpu
from jax.experimental.pallas import tpu_sc as plsc

N = 10000
NP = 10240
E = 320000
EP = 323584
CH = 158
CHD = 79
TRASH = 10200
D_IN = 128
HID = 128
EMB = 256
NG = 16
NF = 4
B = 256
R = 512
NR = NP // R

_MESH = dict(core_axis_name="c", subcore_axis_name="s")


def _deg_sc(dstdeg):
    @functools.partial(
        pl.kernel,
        out_type=jax.ShapeDtypeStruct((2, NP, 128), jnp.float32),
        mesh=plsc.VectorSubcoreMesh(**_MESH),
        scratch_types=[
            pltpu.VMEM((CHD, 128), jnp.int32),
            pltpu.VMEM((128, 128), jnp.float32),
            pltpu.VMEM_SHARED((NP, 128), jnp.float32),
        ],
    )
    def k(dst_hbm, out_hbm, idx_v, ones_v, acc):
        c = lax.axis_index("c")
        s = lax.axis_index("s")
        rpt = NP // 16
        r0 = s * rpt

        @pl.loop(0, 128)
        def _(i):
            @pl.loop(0, 8)
            def _(q):
                ones_v[i, pl.ds(q * 16, 16)] = jnp.zeros((16,), jnp.float32)

        @pl.loop(0, rpt // 128)
        def _(b):
            pltpu.sync_copy(ones_v, acc.at[pl.ds(r0 + b * 128, 128)])

        @pl.loop(0, 128)
        def _(i):
            @pl.loop(0, 8)
            def _(q):
                ones_v[i, pl.ds(q * 16, 16)] = jnp.ones((16,), jnp.float32)

        pltpu.sync_copy(dst_hbm.at[c].at[s], idx_v)
        plsc.subcore_barrier()

        @pl.loop(0, CHD)
        def _(j):
            pltpu.sync_copy(ones_v, acc.at[idx_v.at[j]], add=True)

        plsc.subcore_barrier()
        pltpu.sync_copy(acc.at[pl.ds(r0, rpt)], out_hbm.at[c].at[pl.ds(r0, rpt)])

    return k(dstdeg)


def _mp1_sc(p, srcg, dstg):
    dh = HID

    @functools.partial(
        pl.kernel,
        out_type=jax.ShapeDtypeStruct((2, NP, dh), jnp.float32),
        mesh=plsc.VectorSubcoreMesh(**_MESH),
        scratch_types=[
            pltpu.VMEM((CHD, 128), jnp.int32),
            pltpu.VMEM((CHD, 128), jnp.int32),
            pltpu.VMEM((128, dh), jnp.float32),
            pltpu.VMEM_SHARED((NP, dh), jnp.float32),
        ],
    )
    def k(p_hbm, src_hbm, dst_hbm, out_hbm, src_v, dst_v, rows, acc):
        c = lax.axis_index("c")
        s = lax.axis_index("s")
        rpt = NP // 16
        r0 = s * rpt

        @pl.when(c == 0)
        def _():
            pltpu.sync_copy(p_hbm.at[pl.ds(r0, rpt)], acc.at[pl.ds(r0, rpt)])

        @pl.when(c != 0)
        def _():
            @pl.loop(0, 128)
            def _(i):
                @pl.loop(0, dh // 16)
                def _(q):
                    rows[i, pl.ds(q * 16, 16)] = jnp.zeros((16,), jnp.float32)

            @pl.loop(0, rpt // 128)
            def _(b):
                pltpu.sync_copy(rows, acc.at[pl.ds(r0 + b * 128, 128)])

        pltpu.sync_copy(src_hbm.at[c].at[s], src_v)
        pltpu.sync_copy(dst_hbm.at[c].at[s], dst_v)
        plsc.subcore_barrier()

        @pl.loop(0, CHD)
        def _(j):
            pltpu.sync_copy(p_hbm.at[src_v.at[j]], rows)
            pltpu.sync_copy(rows, acc.at[dst_v.at[j]], add=True)

        plsc.subcore_barrier()
        pltpu.sync_copy(acc.at[pl.ds(r0, rpt)], out_hbm.at[c].at[pl.ds(r0, rpt)])

    return k(p, srcg, dstg)


def _mp2_sc(p0, p1, prev, srcg, dstg, dh):
    have_prev = prev is not None

    @functools.partial(
        pl.kernel,
        out_type=jax.ShapeDtypeStruct((2, NP, dh), jnp.float32),
        mesh=plsc.VectorSubcoreMesh(**_MESH),
        scratch_types=[
            pltpu.VMEM((CHD, 128), jnp.int32),
            pltpu.VMEM((CHD, 128), jnp.int32),
            pltpu.VMEM((128, dh), jnp.float32),
            pltpu.VMEM_SHARED((NP, dh), jnp.float32),
        ],
    )
    def k(p0_hbm, p1_hbm, *rest):
        if have_prev:
            (prev_hbm, src_hbm, dst_hbm, out_hbm,
             src_v, dst_v, rows, acc) = rest
        else:
            (src_hbm, dst_hbm, out_hbm,
             src_v, dst_v, rows, acc) = rest
        c = lax.axis_index("c")
        s = lax.axis_index("s")
        rpt = NP // 16
        sl = pl.ds(s * rpt, rpt)

        if have_prev:
            pltpu.sync_copy(prev_hbm.at[c].at[sl], acc.at[sl])
        else:
            @pl.when(c == 0)
            def _():
                pltpu.sync_copy(p0_hbm.at[sl], acc.at[sl])

            @pl.when(c != 0)
            def _():
                pltpu.sync_copy(p1_hbm.at[sl], acc.at[sl])

        pltpu.sync_copy(src_hbm.at[s], src_v)
        pltpu.sync_copy(dst_hbm.at[s], dst_v)
        plsc.subcore_barrier()

        @pl.loop(0, CHD)
        def _(j):
            @pl.when(c == 0)
            def _():
                pltpu.sync_copy(p0_hbm.at[src_v.at[j]], rows)

            @pl.when(c != 0)
            def _():
                pltpu.sync_copy(p1_hbm.at[src_v.at[j]], rows)

            pltpu.sync_copy(rows, acc.at[dst_v.at[j]], add=True)

        plsc.subcore_barrier()
        pltpu.sync_copy(acc.at[sl], out_hbm.at[c].at[sl])

    if have_prev:
        return k(p0, p1, prev, srcg, dstg)
    return k(p0, p1, srcg, dstg)


def _a_body(deg_ref, x_ref, w_ref, p_ref, dinv_ref):
    deg = deg_ref[0, :, 0] + deg_ref[1, :, 0] + 1.0
    dinv = lax.rsqrt(jnp.maximum(deg, 1.0))[:, None]
    dinv_ref[...] = dinv
    u = jnp.dot(x_ref[...], w_ref[...], preferred_element_type=jnp.float32)
    p_ref[...] = u * dinv


def _a_tc(degacc, x_pad, W1):
    return pl.pallas_call(
        _a_body,
        grid=(NR,),
        in_specs=[
            pl.BlockSpec((2, R, 128), lambda i: (0, i, 0)),
            pl.BlockSpec((R, D_IN), lambda i: (i, 0)),
            pl.BlockSpec((D_IN, HID), lambda i: (0, 0)),
        ],
        out_specs=[
            pl.BlockSpec((R, HID), lambda i: (i, 0)),
            pl.BlockSpec((R, 1), lambda i: (i, 0)),
        ],
        out_shape=[
            jax.ShapeDtypeStruct((NP, HID), jnp.float32),
            jax.ShapeDtypeStruct((NP, 1), jnp.float32),
        ],
    )(degacc, x_pad, W1)


def _b_body(m_ref, dinv_ref, b1_ref, w2_ref, p2a_ref, p2b_ref):
    dinv = dinv_ref[...]
    h = m_ref[0] + m_ref[1]
    h1 = jnp.maximum(h * dinv + b1_ref[...], 0.0)
    p2 = jnp.dot(h1, w2_ref[...], preferred_element_type=jnp.float32) * dinv
    p2a_ref[...] = p2[:, :EMB // 2]
    p2b_ref[...] = p2[:, EMB // 2:]


def _b_tc(m1, dinv, b1r, W2):
    return pl.pallas_call(
        _b_body,
        grid=(NR,),
        in_specs=[
            pl.BlockSpec((2, R, HID), lambda i: (0, i, 0)),
            pl.BlockSpec((R, 1), lambda i: (i, 0)),
            pl.BlockSpec((1, HID), lambda i: (0, 0)),
            pl.BlockSpec((HID, EMB), lambda i: (0, 0)),
        ],
        out_specs=[
            pl.BlockSpec((R, EMB // 2), lambda i: (i, 0)),
            pl.BlockSpec((R, EMB // 2), lambda i: (i, 0)),
        ],
        out_shape=[
            jax.ShapeDtypeStruct((NP, EMB // 2), jnp.float32),
            jax.ShapeDtypeStruct((NP, EMB // 2), jnp.float32),
        ],
    )(m1, dinv, b1r, W2)


def _c_body(m_ref, dinv_ref, b2_ref, batch_ref, wg_ref, bg_ref,
            wf_ref, bfam_ref, emb_ref, gl_ref, fl_ref, sums, cnts):
    i = pl.program_id(0)

    @pl.when(i == 0)
    def _():
        sums[...] = jnp.zeros_like(sums)
        cnts[...] = jnp.zeros_like(cnts)

    dinv = dinv_ref[...]
    h2 = jnp.concatenate([m_ref[0], m_ref[1]], axis=1) * dinv + b2_ref[...]
    bb = batch_ref[0]
    gid = lax.broadcasted_iota(jnp.int32, (B, R), 0)
    onehot = (bb == gid).astype(jnp.float32)
    sums[...] += jnp.dot(onehot, h2, preferred_element_type=jnp.float32)
    cnts[...] += jnp.sum(onehot, axis=1, keepdims=True)

    @pl.when(i == pl.num_programs(0) - 1)
    def _():
        emb = sums[...] / jnp.maximum(cnts[...], 1.0)
        emb_ref[...] = emb
        gl_ref[...] = (jnp.dot(emb, wg_ref[...],
                               preferred_element_type=jnp.float32)
                       + bg_ref[...])
        fl_ref[...] = (jnp.dot(emb, wf_ref[...],
                               preferred_element_type=jnp.float32)
                       + bfam_ref[...])


def _c_tc(m2, dinv, b2r, batch_g, Wg, bgr, famW, fambr):
    return pl.pallas_call(
        _c_body,
        grid=(NR,),
        in_specs=[
            pl.BlockSpec((2, R, EMB // 2), lambda i: (0, i, 0)),
            pl.BlockSpec((R, 1), lambda i: (i, 0)),
            pl.BlockSpec((1, EMB), lambda i: (0, 0)),
            pl.BlockSpec((1, 1, R), lambda i: (i, 0, 0)),
            pl.BlockSpec((EMB, NG), lambda i: (0, 0)),
            pl.BlockSpec((1, NG), lambda i: (0, 0)),
            pl.BlockSpec((EMB, NG * NF), lambda i: (0, 0)),
            pl.BlockSpec((1, NG * NF), lambda i: (0, 0)),
        ],
        out_specs=[
            pl.BlockSpec((B, EMB), lambda i: (0, 0)),
            pl.BlockSpec((B, NG), lambda i: (0, 0)),
            pl.BlockSpec((B, NG * NF), lambda i: (0, 0)),
        ],
        out_shape=[
            jax.ShapeDtypeStruct((B, EMB), jnp.float32),
            jax.ShapeDtypeStruct((B, NG), jnp.float32),
            jax.ShapeDtypeStruct((B, NG * NF), jnp.float32),
        ],
        scratch_shapes=[
            pltpu.VMEM((B, EMB), jnp.float32),
            pltpu.VMEM((B, 1), jnp.float32),
        ],
    )(m2, dinv, b2r, batch_g, Wg, bgr, famW, fambr)


def kernel(x, edge_index, batch, W1, b1, W2, b2, Wg, bg, Wf, bf):
    x_pad = jnp.pad(x, ((0, NP - N), (0, 0)))
    src_pad = jnp.pad(edge_index[0].astype(jnp.int32), (0, EP - E))
    dst_pad = jnp.pad(edge_index[1].astype(jnp.int32), (0, EP - E),
                      constant_values=TRASH)
    srcdeg = src_pad.reshape(2, 16, CHD, 128)
    dstdeg = dst_pad.reshape(2, 16, CHD, 128)
    batch_g = jnp.pad(batch.astype(jnp.int32), (0, NP - N),
                      constant_values=B + 7).reshape(NR, 1, R)
    famW = Wf.transpose(1, 0, 2).reshape(EMB, NG * NF)

    degacc = _deg_sc(dstdeg)
    p1, dinv = _a_tc(degacc, x_pad, W1)
    m1 = _mp1_sc(p1, srcdeg, dstdeg)
    p2a, p2b = _b_tc(m1, dinv, b1.reshape(1, HID), W2)
    m2h = _mp2_sc(p2a, p2b, None, srcdeg[0], dstdeg[0], EMB // 2)
    m2 = _mp2_sc(p2a, p2b, m2h, srcdeg[1], dstdeg[1], EMB // 2)
    emb, gl, flf = _c_tc(m2, dinv, b2.reshape(1, EMB), batch_g, Wg,
                         bg.reshape(1, NG), famW, bf.reshape(1, NG * NF))
    fl = flf.reshape(B, NG, NF).transpose(1, 0, 2)
    return emb, gl, fl


# speedup vs baseline: 10.8637x; 1.0226x over previous
import functools

import jax
import jax.numpy as jnp
from jax import lax
from jax.experimental import pallas as pl
from jax.experimental.pallas import tpu as pltpu
from jax.experimental.pallas import tpu_sc as plsc

N = 10000
NP = 10240
E = 320000
EP = 323584
CH = 158
CHD = 79
TRASH = 10200
D_IN = 128
HID = 128
EMB = 256
NG = 16
NF = 4
B = 256
R = 512
NR = NP // R

_MESH = dict(core_axis_name="c", subcore_axis_name="s")


def _deg_sc(dstdeg):
    @functools.partial(
        pl.kernel,
        out_type=jax.ShapeDtypeStruct((2, NP, 128), jnp.float32),
        mesh=plsc.VectorSubcoreMesh(**_MESH),
        scratch_types=[
            pltpu.VMEM((CHD, 128), jnp.int32),
            pltpu.VMEM((128, 128), jnp.float32),
            pltpu.VMEM_SHARED((NP, 128), jnp.float32),
        ],
    )
    def k(dst_hbm, out_hbm, idx_v, ones_v, acc):
        c = lax.axis_index("c")
        s = lax.axis_index("s")
        rpt = NP // 16
        r0 = s * rpt

        @pl.loop(0, 128)
        def _(i):
            @pl.loop(0, 8)
            def _(q):
                ones_v[i, pl.ds(q * 16, 16)] = jnp.zeros((16,), jnp.float32)

        @pl.loop(0, rpt // 128)
        def _(b):
            pltpu.sync_copy(ones_v, acc.at[pl.ds(r0 + b * 128, 128)])

        @pl.loop(0, 128)
        def _(i):
            @pl.loop(0, 8)
            def _(q):
                ones_v[i, pl.ds(q * 16, 16)] = jnp.ones((16,), jnp.float32)

        pltpu.sync_copy(dst_hbm.at[c].at[s], idx_v)
        plsc.subcore_barrier()

        @pl.loop(0, CHD)
        def _(j):
            pltpu.sync_copy(ones_v, acc.at[idx_v.at[j]], add=True)

        plsc.subcore_barrier()
        pltpu.sync_copy(acc.at[pl.ds(r0, rpt)], out_hbm.at[c].at[pl.ds(r0, rpt)])

    return k(dstdeg)


def _mp1_sc(p, srcg, dstg):
    dh = HID

    @functools.partial(
        pl.kernel,
        out_type=jax.ShapeDtypeStruct((2, NP, dh), jnp.float32),
        mesh=plsc.VectorSubcoreMesh(**_MESH),
        scratch_types=[
            pltpu.VMEM((CHD, 128), jnp.int32),
            pltpu.VMEM((CHD, 128), jnp.int32),
            pltpu.VMEM((128, dh), jnp.float32),
            pltpu.VMEM_SHARED((NP, dh), jnp.float32),
        ],
    )
    def k(p_hbm, src_hbm, dst_hbm, out_hbm, src_v, dst_v, rows, acc):
        c = lax.axis_index("c")
        s = lax.axis_index("s")
        rpt = NP // 16
        r0 = s * rpt

        @pl.loop(0, 128)
        def _(i):
            @pl.loop(0, dh // 16)
            def _(q):
                rows[i, pl.ds(q * 16, 16)] = jnp.zeros((16,), jnp.float32)

        @pl.loop(0, rpt // 128)
        def _(b):
            pltpu.sync_copy(rows, acc.at[pl.ds(r0 + b * 128, 128)])

        pltpu.sync_copy(src_hbm.at[c].at[s], src_v)
        pltpu.sync_copy(dst_hbm.at[c].at[s], dst_v)
        plsc.subcore_barrier()

        @pl.loop(0, CHD)
        def _(j):
            pltpu.sync_copy(p_hbm.at[src_v.at[j]], rows)
            pltpu.sync_copy(rows, acc.at[dst_v.at[j]], add=True)

        plsc.subcore_barrier()
        pltpu.sync_copy(acc.at[pl.ds(r0, rpt)], out_hbm.at[c].at[pl.ds(r0, rpt)])

    return k(p, srcg, dstg)


def _mp2_sc(p0, p1, srcg, dstg, dh):
    @functools.partial(
        pl.kernel,
        out_type=jax.ShapeDtypeStruct((2, NP, dh), jnp.float32),
        mesh=plsc.VectorSubcoreMesh(**_MESH),
        scratch_types=[
            pltpu.VMEM((CHD, 128), jnp.int32),
            pltpu.VMEM((CHD, 128), jnp.int32),
            pltpu.VMEM((128, dh), jnp.float32),
            pltpu.VMEM_SHARED((NP, dh), jnp.float32),
        ],
    )
    def k(p0_hbm, p1_hbm, src_hbm, dst_hbm, out_hbm, src_v, dst_v, rows, acc):
        c = lax.axis_index("c")
        s = lax.axis_index("s")
        rpt = NP // 16
        sl = pl.ds(s * rpt, rpt)

        @pl.loop(0, 128)
        def _(i):
            @pl.loop(0, dh // 16)
            def _(q):
                rows[i, pl.ds(q * 16, 16)] = jnp.zeros((16,), jnp.float32)

        @pl.loop(0, rpt // 128)
        def _(b):
            pltpu.sync_copy(rows, acc.at[pl.ds(s * rpt + b * 128, 128)])

        pltpu.sync_copy(src_hbm.at[s], src_v)
        pltpu.sync_copy(dst_hbm.at[s], dst_v)
        plsc.subcore_barrier()

        @pl.loop(0, CHD)
        def _(j):
            @pl.when(c == 0)
            def _():
                pltpu.sync_copy(p0_hbm.at[src_v.at[j]], rows)

            @pl.when(c != 0)
            def _():
                pltpu.sync_copy(p1_hbm.at[src_v.at[j]], rows)

            pltpu.sync_copy(rows, acc.at[dst_v.at[j]], add=True)

        plsc.subcore_barrier()
        pltpu.sync_copy(acc.at[sl], out_hbm.at[c].at[sl])

    return k(p0, p1, srcg, dstg)


def _a_body(deg_ref, x_ref, w_ref, p_ref, dinv_ref):
    deg = deg_ref[0, :, 0] + deg_ref[1, :, 0] + 1.0
    dinv = lax.rsqrt(jnp.maximum(deg, 1.0))[:, None]
    dinv_ref[...] = dinv
    u = jnp.dot(x_ref[...], w_ref[...], preferred_element_type=jnp.float32)
    p_ref[...] = u * dinv


def _a_tc(degacc, x_pad, W1):
    return pl.pallas_call(
        _a_body,
        grid=(NR,),
        in_specs=[
            pl.BlockSpec((2, R, 128), lambda i: (0, i, 0)),
            pl.BlockSpec((R, D_IN), lambda i: (i, 0)),
            pl.BlockSpec((D_IN, HID), lambda i: (0, 0)),
        ],
        out_specs=[
            pl.BlockSpec((R, HID), lambda i: (i, 0)),
            pl.BlockSpec((R, 1), lambda i: (i, 0)),
        ],
        out_shape=[
            jax.ShapeDtypeStruct((NP, HID), jnp.float32),
            jax.ShapeDtypeStruct((NP, 1), jnp.float32),
        ],
    )(degacc, x_pad, W1)


def _b_body(m_ref, p1_ref, dinv_ref, b1_ref, w2_ref, p2a_ref, p2b_ref):
    dinv = dinv_ref[...]
    h = m_ref[0] + m_ref[1] + p1_ref[...]
    h1 = jnp.maximum(h * dinv + b1_ref[...], 0.0)
    p2 = jnp.dot(h1, w2_ref[...], preferred_element_type=jnp.float32) * dinv
    p2a_ref[...] = p2[:, :EMB // 2]
    p2b_ref[...] = p2[:, EMB // 2:]


def _b_tc(m1, p1, dinv, b1r, W2):
    return pl.pallas_call(
        _b_body,
        grid=(NR,),
        in_specs=[
            pl.BlockSpec((2, R, HID), lambda i: (0, i, 0)),
            pl.BlockSpec((R, HID), lambda i: (i, 0)),
            pl.BlockSpec((R, 1), lambda i: (i, 0)),
            pl.BlockSpec((1, HID), lambda i: (0, 0)),
            pl.BlockSpec((HID, EMB), lambda i: (0, 0)),
        ],
        out_specs=[
            pl.BlockSpec((R, EMB // 2), lambda i: (i, 0)),
            pl.BlockSpec((R, EMB // 2), lambda i: (i, 0)),
        ],
        out_shape=[
            jax.ShapeDtypeStruct((NP, EMB // 2), jnp.float32),
            jax.ShapeDtypeStruct((NP, EMB // 2), jnp.float32),
        ],
    )(m1, p1, dinv, b1r, W2)


def _c_body(mx_ref, my_ref, p2a_ref, p2b_ref, dinv_ref, b2_ref, batch_ref,
            wg_ref, bg_ref, wf_ref, bfam_ref, emb_ref, gl_ref, fl_ref,
            sums, cnts):
    i = pl.program_id(0)

    @pl.when(i == 0)
    def _():
        sums[...] = jnp.zeros_like(sums)
        cnts[...] = jnp.zeros_like(cnts)

    dinv = dinv_ref[...]
    m0 = mx_ref[0] + my_ref[0] + p2a_ref[...]
    m1 = mx_ref[1] + my_ref[1] + p2b_ref[...]
    h2 = jnp.concatenate([m0, m1], axis=1) * dinv + b2_ref[...]
    bb = batch_ref[0]
    gid = lax.broadcasted_iota(jnp.int32, (B, R), 0)
    onehot = (bb == gid).astype(jnp.float32)
    sums[...] += jnp.dot(onehot, h2, preferred_element_type=jnp.float32)
    cnts[...] += jnp.sum(onehot, axis=1, keepdims=True)

    @pl.when(i == pl.num_programs(0) - 1)
    def _():
        emb = sums[...] / jnp.maximum(cnts[...], 1.0)
        emb_ref[...] = emb
        gl_ref[...] = (jnp.dot(emb, wg_ref[...],
                               preferred_element_type=jnp.float32)
                       + bg_ref[...])
        fl_ref[...] = (jnp.dot(emb, wf_ref[...],
                               preferred_element_type=jnp.float32)
                       + bfam_ref[...])


def _c_tc(m2x, m2y, p2a, p2b, dinv, b2r, batch_g, Wg, bgr, famW, fambr):
    return pl.pallas_call(
        _c_body,
        grid=(NR,),
        in_specs=[
            pl.BlockSpec((2, R, EMB // 2), lambda i: (0, i, 0)),
            pl.BlockSpec((2, R, EMB // 2), lambda i: (0, i, 0)),
            pl.BlockSpec((R, EMB // 2), lambda i: (i, 0)),
            pl.BlockSpec((R, EMB // 2), lambda i: (i, 0)),
            pl.BlockSpec((R, 1), lambda i: (i, 0)),
            pl.BlockSpec((1, EMB), lambda i: (0, 0)),
            pl.BlockSpec((1, 1, R), lambda i: (i, 0, 0)),
            pl.BlockSpec((EMB, NG), lambda i: (0, 0)),
            pl.BlockSpec((1, NG), lambda i: (0, 0)),
            pl.BlockSpec((EMB, NG * NF), lambda i: (0, 0)),
            pl.BlockSpec((1, NG * NF), lambda i: (0, 0)),
        ],
        out_specs=[
            pl.BlockSpec((B, EMB), lambda i: (0, 0)),
            pl.BlockSpec((B, NG), lambda i: (0, 0)),
            pl.BlockSpec((B, NG * NF), lambda i: (0, 0)),
        ],
        out_shape=[
            jax.ShapeDtypeStruct((B, EMB), jnp.float32),
            jax.ShapeDtypeStruct((B, NG), jnp.float32),
            jax.ShapeDtypeStruct((B, NG * NF), jnp.float32),
        ],
        scratch_shapes=[
            pltpu.VMEM((B, EMB), jnp.float32),
            pltpu.VMEM((B, 1), jnp.float32),
        ],
    )(m2x, m2y, p2a, p2b, dinv, b2r, batch_g, Wg, bgr, famW, fambr)


def kernel(x, edge_index, batch, W1, b1, W2, b2, Wg, bg, Wf, bf):
    x_pad = jnp.pad(x, ((0, NP - N), (0, 0)))
    src_pad = jnp.pad(edge_index[0].astype(jnp.int32), (0, EP - E))
    dst_pad = jnp.pad(edge_index[1].astype(jnp.int32), (0, EP - E),
                      constant_values=TRASH)
    srcdeg = src_pad.reshape(2, 16, CHD, 128)
    dstdeg = dst_pad.reshape(2, 16, CHD, 128)
    batch_g = jnp.pad(batch.astype(jnp.int32), (0, NP - N),
                      constant_values=B + 7).reshape(NR, 1, R)
    famW = Wf.transpose(1, 0, 2).reshape(EMB, NG * NF)

    degacc = _deg_sc(dstdeg)
    p1, dinv = _a_tc(degacc, x_pad, W1)
    m1 = _mp1_sc(p1, srcdeg, dstdeg)
    p2a, p2b = _b_tc(m1, p1, dinv, b1.reshape(1, HID), W2)
    m2x = _mp2_sc(p2a, p2b, srcdeg[0], dstdeg[0], EMB // 2)
    m2y = _mp2_sc(p2a, p2b, srcdeg[1], dstdeg[1], EMB // 2)
    emb, gl, flf = _c_tc(m2x, m2y, p2a, p2b, dinv, b2.reshape(1, EMB),
                         batch_g, Wg, bg.reshape(1, NG), famW,
                         bf.reshape(1, NG * NF))
    fl = flf.reshape(B, NG, NF).transpose(1, 0, 2)
    return emb, gl, fl


# speedup vs baseline: 10.8682x; 1.0004x over previous
import functools

import jax
import jax.numpy as jnp
from jax import lax
from jax.experimental import pallas as pl
from jax.experimental.pallas import tpu as pltpu
from jax.experimental.pallas import tpu_sc as plsc

N = 10000
NP = 10240
E = 320000
EP = 323584
CH = 158
CHD = 79
TRASH = 10200
D_IN = 128
HID = 128
EMB = 256
NG = 16
NF = 4
B = 256
R = 512
NR = NP // R

_MESH = dict(core_axis_name="c", subcore_axis_name="s")


def _deg_sc(dstdeg):
    @functools.partial(
        pl.kernel,
        out_type=jax.ShapeDtypeStruct((2, NP, 128), jnp.float32),
        mesh=plsc.VectorSubcoreMesh(**_MESH),
        scratch_types=[
            pltpu.VMEM((CHD, 128), jnp.int32),
            pltpu.VMEM((128, 128), jnp.float32),
            pltpu.VMEM_SHARED((NP, 128), jnp.float32),
        ],
    )
    def k(dst_hbm, out_hbm, idx_v, ones_v, acc):
        c = lax.axis_index("c")
        s = lax.axis_index("s")
        rpt = NP // 16
        r0 = s * rpt

        @pl.loop(0, 128)
        def _(i):
            @pl.loop(0, 8)
            def _(q):
                ones_v[i, pl.ds(q * 16, 16)] = jnp.zeros((16,), jnp.float32)

        @pl.loop(0, rpt // 128)
        def _(b):
            pltpu.sync_copy(ones_v, acc.at[pl.ds(r0 + b * 128, 128)])

        @pl.loop(0, 128)
        def _(i):
            @pl.loop(0, 8)
            def _(q):
                ones_v[i, pl.ds(q * 16, 16)] = jnp.ones((16,), jnp.float32)

        pltpu.sync_copy(dst_hbm.at[c].at[s], idx_v)
        plsc.subcore_barrier()

        @pl.loop(0, CHD)
        def _(j):
            pltpu.sync_copy(ones_v, acc.at[idx_v.at[j]], add=True)

        plsc.subcore_barrier()
        pltpu.sync_copy(acc.at[pl.ds(r0, rpt)], out_hbm.at[c].at[pl.ds(r0, rpt)])

    return k(dstdeg)


def _mp1_sc(p, srcg, dstg):
    dh = HID

    @functools.partial(
        pl.kernel,
        out_type=jax.ShapeDtypeStruct((2, NP, dh), jnp.float32),
        mesh=plsc.VectorSubcoreMesh(**_MESH),
        scratch_types=[
            pltpu.VMEM((CHD, 128), jnp.int32),
            pltpu.VMEM((CHD, 128), jnp.int32),
            pltpu.VMEM((128, dh), jnp.float32),
            pltpu.VMEM_SHARED((NP, dh), jnp.float32),
        ],
    )
    def k(p_hbm, src_hbm, dst_hbm, out_hbm, src_v, dst_v, rows, acc):
        c = lax.axis_index("c")
        s = lax.axis_index("s")
        rpt = NP // 16
        r0 = s * rpt

        @pl.loop(0, 128)
        def _(i):
            @pl.loop(0, dh // 16)
            def _(q):
                rows[i, pl.ds(q * 16, 16)] = jnp.zeros((16,), jnp.float32)

        @pl.loop(0, rpt // 128)
        def _(b):
            pltpu.sync_copy(rows, acc.at[pl.ds(r0 + b * 128, 128)])

        pltpu.sync_copy(src_hbm.at[c].at[s], src_v)
        pltpu.sync_copy(dst_hbm.at[c].at[s], dst_v)
        plsc.subcore_barrier()

        @pl.loop(0, CHD)
        def _(j):
            pltpu.sync_copy(p_hbm.at[src_v.at[j]], rows)
            pltpu.sync_copy(rows, acc.at[dst_v.at[j]], add=True)

        plsc.subcore_barrier()
        pltpu.sync_copy(acc.at[pl.ds(r0, rpt)], out_hbm.at[c].at[pl.ds(r0, rpt)])

    return k(p, srcg, dstg)


def _mp2_sc(p0, p1, srcg, dstg, dh):
    @functools.partial(
        pl.kernel,
        out_type=jax.ShapeDtypeStruct((2, NP, dh), jnp.float32),
        mesh=plsc.VectorSubcoreMesh(**_MESH),
        scratch_types=[
            pltpu.VMEM((CHD, 128), jnp.int32),
            pltpu.VMEM((CHD, 128), jnp.int32),
            pltpu.VMEM((128, dh), jnp.float32),
            pltpu.VMEM_SHARED((NP, dh), jnp.float32),
        ],
    )
    def k(p0_hbm, p1_hbm, src_hbm, dst_hbm, out_hbm, src_v, dst_v, rows, acc):
        c = lax.axis_index("c")
        s = lax.axis_index("s")
        rpt = NP // 16
        sl = pl.ds(s * rpt, rpt)

        @pl.loop(0, 128)
        def _(i):
            @pl.loop(0, dh // 16)
            def _(q):
                rows[i, pl.ds(q * 16, 16)] = jnp.zeros((16,), jnp.float32)

        @pl.loop(0, rpt // 128)
        def _(b):
            pltpu.sync_copy(rows, acc.at[pl.ds(s * rpt + b * 128, 128)])

        pltpu.sync_copy(src_hbm.at[s], src_v)
        pltpu.sync_copy(dst_hbm.at[s], dst_v)
        plsc.subcore_barrier()

        @pl.loop(0, CHD)
        def _(j):
            @pl.when(c == 0)
            def _():
                pltpu.sync_copy(p0_hbm.at[src_v.at[j]], rows)

            @pl.when(c != 0)
            def _():
                pltpu.sync_copy(p1_hbm.at[src_v.at[j]], rows)

            pltpu.sync_copy(rows, acc.at[dst_v.at[j]], add=True)

        plsc.subcore_barrier()
        pltpu.sync_copy(acc.at[sl], out_hbm.at[c].at[sl])

    return k(p0, p1, srcg, dstg)


def _a_body(deg_ref, x_ref, w_ref, p_ref, dinv_ref):
    deg = deg_ref[0, :, 0] + deg_ref[1, :, 0] + 1.0
    dinv = lax.rsqrt(jnp.maximum(deg, 1.0))[:, None]
    dinv_ref[...] = dinv
    u = jnp.dot(x_ref[...], w_ref[...], preferred_element_type=jnp.float32)
    p_ref[...] = u * dinv


def _a_tc(degacc, x_pad, W1):
    return pl.pallas_call(
        _a_body,
        grid=(NR,),
        in_specs=[
            pl.BlockSpec((2, R, 128), lambda i: (0, i, 0)),
            pl.BlockSpec((R, D_IN), lambda i: (i, 0)),
            pl.BlockSpec((D_IN, HID), lambda i: (0, 0)),
        ],
        out_specs=[
            pl.BlockSpec((R, HID), lambda i: (i, 0)),
            pl.BlockSpec((R, 1), lambda i: (i, 0)),
        ],
        out_shape=[
            jax.ShapeDtypeStruct((NP, HID), jnp.float32),
            jax.ShapeDtypeStruct((NP, 1), jnp.float32),
        ],
    )(degacc, x_pad, W1)


def _b_body(m_ref, p1_ref, dinv_ref, b1_ref, w2_ref, p2a_ref, p2b_ref):
    dinv = dinv_ref[...]
    h = m_ref[0] + m_ref[1] + p1_ref[...]
    h1 = jnp.maximum(h * dinv + b1_ref[...], 0.0)
    p2 = jnp.dot(h1, w2_ref[...], preferred_element_type=jnp.float32) * dinv
    p2a_ref[...] = p2[:, :EMB // 2]
    p2b_ref[...] = p2[:, EMB // 2:]


def _b_tc(m1, p1, dinv, b1r, W2):
    return pl.pallas_call(
        _b_body,
        grid=(NR,),
        in_specs=[
            pl.BlockSpec((2, R, HID), lambda i: (0, i, 0)),
            pl.BlockSpec((R, HID), lambda i: (i, 0)),
            pl.BlockSpec((R, 1), lambda i: (i, 0)),
            pl.BlockSpec((1, HID), lambda i: (0, 0)),
            pl.BlockSpec((HID, EMB), lambda i: (0, 0)),
        ],
        out_specs=[
            pl.BlockSpec((R, EMB // 2), lambda i: (i, 0)),
            pl.BlockSpec((R, EMB // 2), lambda i: (i, 0)),
        ],
        out_shape=[
            jax.ShapeDtypeStruct((NP, EMB // 2), jnp.float32),
            jax.ShapeDtypeStruct((NP, EMB // 2), jnp.float32),
        ],
    )(m1, p1, dinv, b1r, W2)


def _c_body(mx_ref, my_ref, p2a_ref, p2b_ref, dinv_ref, b2_ref, batch_ref,
            wg_ref, bg_ref, wf_ref, bfam_ref, emb_ref, gl_ref, fl_ref,
            sums, cnts):
    i = pl.program_id(0)

    @pl.when(i == 0)
    def _():
        sums[...] = jnp.zeros_like(sums)
        cnts[...] = jnp.zeros_like(cnts)

    dinv = dinv_ref[...]
    m0 = mx_ref[0] + my_ref[0] + p2a_ref[...]
    m1 = mx_ref[1] + my_ref[1] + p2b_ref[...]
    h2 = jnp.concatenate([m0, m1], axis=1) * dinv + b2_ref[...]
    bb = batch_ref[0]
    gid = lax.broadcasted_iota(jnp.int32, (B, R), 0)
    onehot = (bb == gid).astype(jnp.float32)
    sums[...] += jnp.dot(onehot, h2, preferred_element_type=jnp.float32)
    cnts[...] += jnp.sum(onehot, axis=1, keepdims=True)

    @pl.when(i == pl.num_programs(0) - 1)
    def _():
        emb = sums[...] / jnp.maximum(cnts[...], 1.0)
        emb_ref[...] = emb
        gl_ref[...] = (jnp.dot(emb, wg_ref[...],
                               preferred_element_type=jnp.float32)
                       + bg_ref[...])
        fl_ref[...] = (jnp.dot(emb, wf_ref[...],
                               preferred_element_type=jnp.float32)
                       + bfam_ref[...])


def _c_tc(m2x, m2y, p2a, p2b, dinv, b2r, batch_g, Wg, bgr, famW, fambr):
    return pl.pallas_call(
        _c_body,
        grid=(NR,),
        in_specs=[
            pl.BlockSpec((2, R, EMB // 2), lambda i: (0, i, 0)),
            pl.BlockSpec((2, R, EMB // 2), lambda i: (0, i, 0)),
            pl.BlockSpec((R, EMB // 2), lambda i: (i, 0)),
            pl.BlockSpec((R, EMB // 2), lambda i: (i, 0)),
            pl.BlockSpec((R, 1), lambda i: (i, 0)),
            pl.BlockSpec((1, EMB), lambda i: (0, 0)),
            pl.BlockSpec((1, 1, R), lambda i: (i, 0, 0)),
            pl.BlockSpec((EMB, NG), lambda i: (0, 0)),
            pl.BlockSpec((1, NG), lambda i: (0, 0)),
            pl.BlockSpec((EMB, NG * NF), lambda i: (0, 0)),
            pl.BlockSpec((1, NG * NF), lambda i: (0, 0)),
        ],
        out_specs=[
            pl.BlockSpec((B, EMB), lambda i: (0, 0)),
            pl.BlockSpec((B, NG), lambda i: (0, 0)),
            pl.BlockSpec((B, NG * NF), lambda i: (0, 0)),
        ],
        out_shape=[
            jax.ShapeDtypeStruct((B, EMB), jnp.float32),
            jax.ShapeDtypeStruct((B, NG), jnp.float32),
            jax.ShapeDtypeStruct((B, NG * NF), jnp.float32),
        ],
        scratch_shapes=[
            pltpu.VMEM((B, EMB), jnp.float32),
            pltpu.VMEM((B, 1), jnp.float32),
        ],
    )(m2x, m2y, p2a, p2b, dinv, b2r, batch_g, Wg, bgr, famW, fambr)


def kernel(x, edge_index, batch, W1, b1, W2, b2, Wg, bg, Wf, bf):
    x_pad = jnp.pad(x, ((0, NP - N), (0, 0)))
    src_pad = jnp.pad(edge_index[0].astype(jnp.int32), (0, EP - E))
    trash_dst = N + (jnp.arange(EP - E, dtype=jnp.int32) % (NP - N))
    dst_pad = jnp.concatenate([edge_index[1].astype(jnp.int32), trash_dst])
    srcdeg = src_pad.reshape(2, 16, CHD, 128)
    dstdeg = dst_pad.reshape(2, 16, CHD, 128)
    batch_g = jnp.pad(batch.astype(jnp.int32), (0, NP - N),
                      constant_values=B + 7).reshape(NR, 1, R)
    famW = Wf.transpose(1, 0, 2).reshape(EMB, NG * NF)

    degacc = _deg_sc(dstdeg)
    p1, dinv = _a_tc(degacc, x_pad, W1)
    m1 = _mp1_sc(p1, srcdeg, dstdeg)
    p2a, p2b = _b_tc(m1, p1, dinv, b1.reshape(1, HID), W2)
    m2x = _mp2_sc(p2a, p2b, srcdeg[0], dstdeg[0], EMB // 2)
    m2y = _mp2_sc(p2a, p2b, srcdeg[1], dstdeg[1], EMB // 2)
    emb, gl, flf = _c_tc(m2x, m2y, p2a, p2b, dinv, b2.reshape(1, EMB),
                         batch_g, Wg, bg.reshape(1, NG), famW,
                         bf.reshape(1, NG * NF))
    fl = flf.reshape(B, NG, NF).transpose(1, 0, 2)
    return emb, gl, fl
